# baseline (speedup 1.0000x reference)
import jax
import jax.numpy as jnp
from jax import lax
from jax.experimental import pallas as pl
from jax.experimental.pallas import tpu as pltpu

B = 16384
WIN = 48
WOUT = 63
BLK = 2048


def _permute_body(cmap_ref, x_ref, o_ref):
    rows = lax.broadcasted_iota(jnp.int32, (WIN, WOUT), 0)
    g = (rows == cmap_ref[...]).astype(jnp.float32)
    o_ref[...] = lax.dot_general(
        x_ref[...], g, (((1,), (0,)), ((), ())),
        preferred_element_type=jnp.float32,
        precision=lax.Precision.HIGHEST)


def _permute(in2d, cmap):
    return pl.pallas_call(
        _permute_body,
        grid=(B // BLK,),
        in_specs=[
            pl.BlockSpec((1, WOUT), lambda i: (0, 0)),
            pl.BlockSpec((BLK, WIN), lambda i: (i, 0)),
        ],
        out_specs=pl.BlockSpec((BLK, WOUT), lambda i: (i, 0)),
        out_shape=jax.ShapeDtypeStruct((B, WOUT), jnp.float32),
        compiler_params=pltpu.CompilerParams(
            dimension_semantics=("arbitrary",)),
    )(cmap, in2d)


def kernel(joints, indices):
    cmap = (3 * jnp.repeat(indices.astype(jnp.int32), 3)
            + jnp.tile(jnp.arange(3, dtype=jnp.int32), 21)).reshape(1, WOUT)
    out2d = _permute(joints.reshape(B, WIN), cmap)
    return out2d.reshape(B, 21, 3)


# speedup vs baseline: 1.0416x; 1.0416x over previous
import jax
import jax.numpy as jnp
from jax import lax
from jax.experimental import pallas as pl
from jax.experimental.pallas import tpu as pltpu

B = 16384
WIN = 48
WOUT = 63
BLK = 4096


def _permute_body(cmap_ref, x_ref, o_ref):
    rows = lax.broadcasted_iota(jnp.int32, (WIN, WOUT), 0)
    g = (rows == cmap_ref[...]).astype(jnp.float32)
    o_ref[...] = lax.dot_general(
        x_ref[...], g, (((1,), (0,)), ((), ())),
        preferred_element_type=jnp.float32,
        precision=lax.Precision.HIGHEST)


def _permute(in2d, cmap):
    return pl.pallas_call(
        _permute_body,
        grid=(B // BLK,),
        in_specs=[
            pl.BlockSpec((1, WOUT), lambda i: (0, 0)),
            pl.BlockSpec((BLK, WIN), lambda i: (i, 0)),
        ],
        out_specs=pl.BlockSpec((BLK, WOUT), lambda i: (i, 0)),
        out_shape=jax.ShapeDtypeStruct((B, WOUT), jnp.float32),
        compiler_params=pltpu.CompilerParams(
            dimension_semantics=("parallel",)),
    )(cmap, in2d)


def kernel(joints, indices):
    cmap = (3 * jnp.repeat(indices.astype(jnp.int32), 3)
            + jnp.tile(jnp.arange(3, dtype=jnp.int32), 21)).reshape(1, WOUT)
    out2d = _permute(joints.reshape(B, WIN), cmap)
    return out2d.reshape(B, 21, 3)
